# Initial kernel scaffold; baseline (speedup 1.0000x reference)
import jax
import jax.numpy as jnp
from jax.experimental import pallas as pl


def kernel(net_output, feature, target, kidney_deque, background_deque):
    raise NotImplementedError("write your pallas kernel here")


# speedup vs baseline: 2.2612x; 2.2612x over previous
import functools

import jax
import jax.numpy as jnp
from jax.experimental import pallas as pl
from jax.experimental.pallas import tpu as pltpu


def _pred_masks(no_b, tgt_b):
    n0 = no_b[0:1, :]
    n1 = no_b[1:2, :]
    n2 = no_b[2:3, :]
    p0 = (n0 >= n1) & (n0 >= n2)
    p1 = jnp.logical_not(p0) & (n1 >= n2)
    p2 = jnp.logical_not(p0 | p1)
    km = ((tgt_b == 1) & p1).astype(jnp.float32)
    tm = ((tgt_b == 2) & p2).astype(jnp.float32)
    tw = ((tgt_b == 2) & jnp.logical_not(p2)).astype(jnp.float32)
    return km, tm, tw


def _p1_kernel(no_ref, tg_ref, f_ref, vec_ref, cnt_ref, *, batch):
    i = pl.program_id(0)

    @pl.when(i == 0)
    def _init():
        vec_ref[...] = jnp.zeros_like(vec_ref)
        cnt_ref[...] = jnp.zeros_like(cnt_ref)

    lane8 = jax.lax.broadcasted_iota(jnp.int32, (1, 8), 1)
    for b in range(batch):
        km, tm, tw = _pred_masks(no_ref[b], tg_ref[b])
        f = f_ref[b]
        mk = jnp.concatenate([km, tm], axis=0)
        r = jax.lax.dot_general(
            mk, f, (((1,), (1,)), ((), ())),
            preferred_element_type=jnp.float32)
        vec_ref[b:b + 1, :] += r[0:1, :]
        vec_ref[batch + b:batch + b + 1, :] += r[1:2, :]
        t_cnt = jnp.sum(tm)
        tw_cnt = jnp.sum(tw)
        k_cnt = jnp.sum(km)
        row = (t_cnt * (lane8 == 0) + tw_cnt * (lane8 == 1)
               + k_cnt * (lane8 == 2))
        cnt_ref[b:b + 1, :] += row


def _p2_kernel(no_ref, tg_ref, f_ref, vec_ref, cnt_ref, dq_ref, out_ref,
               proto_ref, tvn_ref, w_ref, acc_ref, *, batch, n_total, q):
    i = pl.program_id(0)
    nblocks = pl.num_programs(0)
    inv_n = 1.0 / n_total

    @pl.when(i == 0)
    def _prologue():
        kvm = vec_ref[0:batch, :] * inv_n
        pad = jnp.zeros((16 - q - batch, kvm.shape[1]), jnp.float32)
        proto = jnp.concatenate([dq_ref[...], kvm, pad], axis=0)
        nrm = jnp.sqrt(jnp.sum(proto * proto, axis=1, keepdims=True)) + 1e-8
        proto_ref[...] = proto / nrm
        tvm = vec_ref[batch:2 * batch, :] * inv_n
        tnrm = jnp.sqrt(jnp.sum(tvm * tvm, axis=1, keepdims=True)) + 1e-8
        tvn_ref[...] = tvm / tnrm
        ka0 = cnt_ref[0, 2] > 0.0
        ka1 = cnt_ref[1, 2] > 0.0
        r16 = jax.lax.broadcasted_iota(jnp.int32, (16, 8), 0)
        c16 = jax.lax.broadcasted_iota(jnp.int32, (16, 8), 1)
        w = ((r16 < q) | ((r16 == q) & ka0)
             | ((r16 == q + 1) & (c16 >= 1) & ka1)).astype(jnp.float32)
        w_ref[...] = w
        for j in range(2 * batch):
            acc_ref[j] = 0.0

    for b in range(batch):
        _, _, tw = _pred_masks(no_ref[b], tg_ref[b])
        f = f_ref[b]
        sq = jnp.sum(f * f, axis=0, keepdims=True)
        rn = 1.0 / (jnp.sqrt(sq) + 1e-8)
        dots = jnp.dot(proto_ref[...], f,
                       preferred_element_type=jnp.float32)
        e = jnp.exp(dots * rn) * tw
        colsum = jnp.sum(e, axis=1, keepdims=True)
        expk_b = jnp.sum(colsum * w_ref[:, b:b + 1])
        svec = jnp.dot(tvn_ref[b:b + 1, :], f,
                       preferred_element_type=jnp.float32)
        s_b = jnp.sum(svec * rn * tw)
        acc_ref[b] += s_b
        acc_ref[batch + b] += expk_b

    @pl.when(i == nblocks - 1)
    def _epilogue():
        et = jnp.float32(0.0)
        ek = jnp.float32(0.0)
        any_c = False
        for b in range(batch):
            c_b = (cnt_ref[b, 0] > 0.0) & (cnt_ref[b, 1] > 0.0)
            et = et + jnp.where(c_b, jnp.exp(acc_ref[b]), 0.0)
            ek = ek + jnp.where(c_b, acc_ref[batch + b], 0.0)
            any_c = c_b | any_c
        denom = jnp.where(any_c, ek, 1.0)
        loss = jnp.where(any_c, (-1.0 / batch) * jnp.log(et / denom), 0.0)
        out_ref[0, 0] = loss


@functools.partial(jax.jit, static_argnames=())
def _run(net_output, feature, target, kidney_deque):
    b, c, d, h, w = net_output.shape
    fd = feature.shape[1]
    q = kidney_deque.shape[0]
    n_total = d * h * w
    nb = 8192
    while n_total % nb != 0:
        nb //= 2
    nblocks = n_total // nb

    no = net_output.reshape(b, c, n_total)
    f = feature.reshape(b, fd, n_total)
    tg = target.reshape(b, 1, n_total)

    vec, cnt = pl.pallas_call(
        functools.partial(_p1_kernel, batch=b),
        grid=(nblocks,),
        in_specs=[
            pl.BlockSpec((b, c, nb), lambda i: (0, 0, i)),
            pl.BlockSpec((b, 1, nb), lambda i: (0, 0, i)),
            pl.BlockSpec((b, fd, nb), lambda i: (0, 0, i)),
        ],
        out_specs=[
            pl.BlockSpec((2 * b, fd), lambda i: (0, 0)),
            pl.BlockSpec((b, 8), lambda i: (0, 0)),
        ],
        out_shape=[
            jax.ShapeDtypeStruct((2 * b, fd), jnp.float32),
            jax.ShapeDtypeStruct((b, 8), jnp.float32),
        ],
    )(no, tg, f)

    loss = pl.pallas_call(
        functools.partial(_p2_kernel, batch=b, n_total=n_total, q=q),
        grid=(nblocks,),
        in_specs=[
            pl.BlockSpec((b, c, nb), lambda i: (0, 0, i)),
            pl.BlockSpec((b, 1, nb), lambda i: (0, 0, i)),
            pl.BlockSpec((b, fd, nb), lambda i: (0, 0, i)),
            pl.BlockSpec((2 * b, fd), lambda i: (0, 0)),
            pl.BlockSpec(memory_space=pltpu.SMEM),
            pl.BlockSpec((q, fd), lambda i: (0, 0)),
        ],
        out_specs=pl.BlockSpec(memory_space=pltpu.SMEM),
        out_shape=jax.ShapeDtypeStruct((1, 1), jnp.float32),
        scratch_shapes=[
            pltpu.VMEM((16, fd), jnp.float32),
            pltpu.VMEM((b, fd), jnp.float32),
            pltpu.VMEM((16, 8), jnp.float32),
            pltpu.SMEM((2 * b,), jnp.float32),
        ],
    )(no, tg, f, vec, cnt, kidney_deque)

    return loss[0, 0]


def kernel(net_output, feature, target, kidney_deque, background_deque):
    del background_deque
    return _run(net_output, feature, target, kidney_deque)


# speedup vs baseline: 2.3855x; 1.0549x over previous
import functools

import jax
import jax.numpy as jnp
from jax.experimental import pallas as pl
from jax.experimental.pallas import tpu as pltpu


def _pred_masks(no_b, tgt_b):
    n0 = no_b[0:1, :]
    n1 = no_b[1:2, :]
    n2 = no_b[2:3, :]
    p0 = (n0 >= n1) & (n0 >= n2)
    p1 = jnp.logical_not(p0) & (n1 >= n2)
    p2 = jnp.logical_not(p0 | p1)
    km = ((tgt_b == 1) & p1).astype(jnp.float32)
    tm = ((tgt_b == 2) & p2).astype(jnp.float32)
    tw = ((tgt_b == 2) & jnp.logical_not(p2)).astype(jnp.float32)
    return km, tm, tw


def _fused_kernel(no_ref, tg_ref, f_ref, dq_ref, out_ref,
                  vec_ref, cnt_ref, proto_ref, tvn_ref, w_ref, acc_ref,
                  *, batch, n_total, q):
    ph = pl.program_id(0)
    i = pl.program_id(1)
    nblocks = pl.num_programs(1)
    inv_n = 1.0 / n_total

    @pl.when((ph == 0) & (i == 0))
    def _init():
        vec_ref[...] = jnp.zeros_like(vec_ref)
        for b in range(batch):
            for j in range(3):
                cnt_ref[b, j] = 0.0

    @pl.when(ph == 0)
    def _pass1():
        for b in range(batch):
            km, tm, tw = _pred_masks(no_ref[b], tg_ref[b])
            f = f_ref[b]
            mk = jnp.concatenate([km, tm], axis=0)
            r = jax.lax.dot_general(
                mk, f, (((1,), (1,)), ((), ())),
                preferred_element_type=jnp.float32)
            vec_ref[b:b + 1, :] += r[0:1, :]
            vec_ref[batch + b:batch + b + 1, :] += r[1:2, :]
            cnt_ref[b, 0] += jnp.sum(tm)
            cnt_ref[b, 1] += jnp.sum(tw)
            cnt_ref[b, 2] += jnp.sum(km)

    @pl.when((ph == 1) & (i == 0))
    def _prologue():
        kvm = vec_ref[0:batch, :] * inv_n
        pad = jnp.zeros((16 - q - batch, kvm.shape[1]), jnp.float32)
        proto = jnp.concatenate([dq_ref[...], kvm, pad], axis=0)
        nrm = jnp.sqrt(jnp.sum(proto * proto, axis=1, keepdims=True)) + 1e-8
        proto_ref[...] = proto / nrm
        tvm = vec_ref[batch:2 * batch, :] * inv_n
        tnrm = jnp.sqrt(jnp.sum(tvm * tvm, axis=1, keepdims=True)) + 1e-8
        tvn_ref[...] = tvm / tnrm
        ka0 = cnt_ref[0, 2] > 0.0
        ka1 = cnt_ref[1, 2] > 0.0
        r16 = jax.lax.broadcasted_iota(jnp.int32, (16, 8), 0)
        c16 = jax.lax.broadcasted_iota(jnp.int32, (16, 8), 1)
        w = ((r16 < q) | ((r16 == q) & ka0)
             | ((r16 == q + 1) & (c16 >= 1) & ka1)).astype(jnp.float32)
        w_ref[...] = w
        for j in range(2 * batch):
            acc_ref[j] = 0.0

    @pl.when(ph == 1)
    def _pass2():
        for b in range(batch):
            _, _, tw = _pred_masks(no_ref[b], tg_ref[b])
            f = f_ref[b]
            sq = jnp.sum(f * f, axis=0, keepdims=True)
            rn = 1.0 / (jnp.sqrt(sq) + 1e-8)
            dots = jnp.dot(proto_ref[...], f,
                           preferred_element_type=jnp.float32)
            e = jnp.exp(dots * rn) * tw
            colsum = jnp.sum(e, axis=1, keepdims=True)
            expk_b = jnp.sum(colsum * w_ref[:, b:b + 1])
            svec = jnp.dot(tvn_ref[b:b + 1, :], f,
                           preferred_element_type=jnp.float32)
            s_b = jnp.sum(svec * rn * tw)
            acc_ref[b] += s_b
            acc_ref[batch + b] += expk_b

    @pl.when((ph == 1) & (i == nblocks - 1))
    def _epilogue():
        et = jnp.float32(0.0)
        ek = jnp.float32(0.0)
        any_c = False
        for b in range(batch):
            c_b = (cnt_ref[b, 0] > 0.0) & (cnt_ref[b, 1] > 0.0)
            et = et + jnp.where(c_b, jnp.exp(acc_ref[b]), 0.0)
            ek = ek + jnp.where(c_b, acc_ref[batch + b], 0.0)
            any_c = c_b | any_c
        denom = jnp.where(any_c, ek, 1.0)
        loss = jnp.where(any_c, (-1.0 / batch) * jnp.log(et / denom), 0.0)
        out_ref[0, 0] = loss


@jax.jit
def _run(net_output, feature, target, kidney_deque):
    b, c, d, h, w = net_output.shape
    fd = feature.shape[1]
    q = kidney_deque.shape[0]
    n_total = d * h * w
    nb = 16384
    while n_total % nb != 0:
        nb //= 2
    nblocks = n_total // nb

    no = net_output.reshape(b, c, n_total)
    f = feature.reshape(b, fd, n_total)
    tg = target.reshape(b, 1, n_total)

    loss = pl.pallas_call(
        functools.partial(_fused_kernel, batch=b, n_total=n_total, q=q),
        grid=(2, nblocks),
        in_specs=[
            pl.BlockSpec((b, c, nb), lambda p, i: (0, 0, i)),
            pl.BlockSpec((b, 1, nb), lambda p, i: (0, 0, i)),
            pl.BlockSpec((b, fd, nb), lambda p, i: (0, 0, i)),
            pl.BlockSpec((q, fd), lambda p, i: (0, 0)),
        ],
        out_specs=pl.BlockSpec(memory_space=pltpu.SMEM),
        out_shape=jax.ShapeDtypeStruct((1, 1), jnp.float32),
        scratch_shapes=[
            pltpu.VMEM((2 * b, fd), jnp.float32),
            pltpu.SMEM((b, 3), jnp.float32),
            pltpu.VMEM((16, fd), jnp.float32),
            pltpu.VMEM((b, fd), jnp.float32),
            pltpu.VMEM((16, 8), jnp.float32),
            pltpu.SMEM((2 * b,), jnp.float32),
        ],
    )(no, tg, f, kidney_deque)

    return loss[0, 0]


def kernel(net_output, feature, target, kidney_deque, background_deque):
    del background_deque
    return _run(net_output, feature, target, kidney_deque)
